# baseline (speedup 1.0000x reference)
import jax
import jax.numpy as jnp
from jax import lax
from jax.experimental import pallas as pl
from jax.experimental.pallas import tpu as pltpu
from jax.experimental.pallas import tpu_sc as plsc

N_ENV = 256
BUF = 2048
N_OBS = 64
N_ACT = 16
BATCH = 256

NC = 2
NS = 16
L = 16
NW = NC * NS
E_PER = N_ENV // NW
IDX_CHUNK = 128
NCHUNK = BATCH // IDX_CHUNK


def _body(obs_hbm, act_hbm, rew_hbm, don_hbm, trn_hbm, nxt_hbm, idx_hbm,
          obs_o, nxt_o, act_o, rew_o, don_o, trn_o,
          idx_loc,
          obs_b, nxt_b, act_b, rew_b, don_b, trn_b, sem):
    wid = lax.axis_index("c") * NS + lax.axis_index("s")
    for e in range(E_PER):
        env = wid * E_PER + e
        pltpu.sync_copy(idx_hbm.at[env], idx_loc)
        copies = []
        for c in range(NCHUNK):
            s = pl.ds(c * IDX_CHUNK, IDX_CHUNK)
            ig = idx_loc.at[s]
            copies.append(pltpu.async_copy(
                obs_hbm.at[env].at[ig], obs_b.at[s], sem))
            copies.append(pltpu.async_copy(
                nxt_hbm.at[env].at[ig], nxt_b.at[s], sem))
            copies.append(pltpu.async_copy(
                act_hbm.at[env].at[ig], act_b.at[s], sem))
            copies.append(pltpu.async_copy(
                rew_hbm.at[env].at[ig], rew_b.at[s], sem))
            copies.append(pltpu.async_copy(
                don_hbm.at[env].at[ig], don_b.at[s], sem))
            copies.append(pltpu.async_copy(
                trn_hbm.at[env].at[ig], trn_b.at[s], sem))
        for cp in copies:
            cp.wait()
        rowb = env * BATCH
        pltpu.sync_copy(obs_b, obs_o.at[pl.ds(rowb, BATCH)])
        pltpu.sync_copy(nxt_b, nxt_o.at[pl.ds(rowb, BATCH)])
        pltpu.sync_copy(act_b, act_o.at[pl.ds(rowb, BATCH)])
        pltpu.sync_copy(rew_b, rew_o.at[pl.ds(rowb, BATCH)])
        pltpu.sync_copy(don_b, don_o.at[pl.ds(rowb, BATCH)])
        pltpu.sync_copy(trn_b, trn_o.at[pl.ds(rowb, BATCH)])


def kernel(observations, actions, rewards, dones, truncations,
           next_observations, indices):
    n_env, buf, n_obs = observations.shape
    n_act = actions.shape[-1]
    batch = indices.shape[1]
    idt = dones.dtype

    mesh = plsc.VectorSubcoreMesh(
        core_axis_name="c", subcore_axis_name="s",
        num_cores=NC, num_subcores=NS)
    f = pl.kernel(
        _body,
        out_type=(
            jax.ShapeDtypeStruct((n_env * batch, n_obs), jnp.float32),
            jax.ShapeDtypeStruct((n_env * batch, n_obs), jnp.float32),
            jax.ShapeDtypeStruct((n_env * batch, n_act), jnp.float32),
            jax.ShapeDtypeStruct((n_env * batch,), jnp.float32),
            jax.ShapeDtypeStruct((n_env * batch,), idt),
            jax.ShapeDtypeStruct((n_env * batch,), idt),
        ),
        mesh=mesh,
        compiler_params=pltpu.CompilerParams(use_tc_tiling_on_sc=False),
        scratch_types=[
            pltpu.VMEM((BATCH,), jnp.int32),
            pltpu.VMEM((BATCH, N_OBS), jnp.float32),
            pltpu.VMEM((BATCH, N_OBS), jnp.float32),
            pltpu.VMEM((BATCH, N_ACT), jnp.float32),
            pltpu.VMEM((BATCH,), jnp.float32),
            pltpu.VMEM((BATCH,), idt),
            pltpu.VMEM((BATCH,), idt),
            pltpu.SemaphoreType.DMA,
        ],
    )
    return f(observations, actions, rewards, dones, truncations,
             next_observations, indices.astype(jnp.int32))


# speedup vs baseline: 3.3430x; 3.3430x over previous
import jax
import jax.numpy as jnp
from jax import lax
from jax.experimental import pallas as pl
from jax.experimental.pallas import tpu as pltpu
from jax.experimental.pallas import tpu_sc as plsc

N_ENV = 256
BUF = 2048
N_OBS = 64
N_ACT = 16
BATCH = 256

NC = 2
NS = 16
L = 16
NW = NC * NS
E_PER = N_ENV // NW

SUB = 8
LANES = 128
TCOLS = BUF // LANES
OBS_BANDS = N_OBS // SUB
ACT_BANDS = N_ACT // SUB
HALF = 2
OTC = BATCH // LANES
NB = BATCH // L


def _tiled3(x):
    n_env, t, f = x.shape
    return (x.transpose(0, 2, 1)
             .reshape(n_env, f // SUB, SUB, t // LANES, LANES)
             .transpose(0, 1, 3, 2, 4))


def _tiled2(x):
    a, b = x.shape
    return (x.reshape(a // SUB, SUB, b // LANES, LANES)
             .transpose(0, 2, 1, 3))


def _untile_out(y, n, f):
    return y.transpose(1, 3, 0, 2).reshape(n, f)


def _body(obs_hbm, act_hbm, rew_hbm, don_hbm, trn_hbm, nxt_hbm, idx_hbm,
          obs_o, nxt_o, act_o, rew_o, don_o, trn_o,
          idx_s, rew_s, don_s, trn_s, slab,
          obs_tb, nxt_tb, act_tb, rew_tb, don_tb, trn_tb):
    wid = lax.axis_index("c") * NS + lax.axis_index("s")
    pltpu.sync_copy(idx_hbm.at[wid], idx_s)
    pltpu.sync_copy(rew_hbm.at[wid], rew_s)
    pltpu.sync_copy(don_hbm.at[wid], don_s)
    pltpu.sync_copy(trn_hbm.at[wid], trn_s)

    def blk_idx(e, j):
        tv = idx_s[j >> 3, e, pl.ds((j & 7) * L, L)]
        return tv >> 7, tv & 127

    def env_body(e, _):
        env = wid * E_PER + e

        for src, dst in ((obs_hbm, obs_tb), (nxt_hbm, nxt_tb)):
            for h in range(OBS_BANDS // HALF):
                pltpu.sync_copy(src.at[env].at[pl.ds(h * HALF, HALF)], slab)

                def half_body(j, _, dst=dst, h=h):
                    tcv, lnv = blk_idx(e, j)
                    for fl in range(HALF * SUB):
                        bv = jnp.full((L,), fl >> 3, jnp.int32)
                        sv = jnp.full((L,), fl & 7, jnp.int32)
                        g = plsc.load_gather(slab, [bv, tcv, sv, lnv])
                        dst[h * HALF + (fl >> 3), j >> 3, fl & 7,
                            pl.ds((j & 7) * L, L)] = g
                    return 0

                lax.fori_loop(0, NB, half_body, 0)

        pltpu.sync_copy(act_hbm.at[env], slab)
        ev = jnp.full((L,), e, jnp.int32)

        def act_body(j, _):
            tcv, lnv = blk_idx(e, j)
            for fl in range(N_ACT):
                bv = jnp.full((L,), fl >> 3, jnp.int32)
                sv = jnp.full((L,), fl & 7, jnp.int32)
                g = plsc.load_gather(slab, [bv, tcv, sv, lnv])
                act_tb[fl >> 3, j >> 3, fl & 7, pl.ds((j & 7) * L, L)] = g
            s = pl.ds(j * L, L)
            rew_tb[s] = plsc.load_gather(rew_s, [tcv, ev, lnv])
            don_tb[s] = plsc.load_gather(don_s, [tcv, ev, lnv])
            trn_tb[s] = plsc.load_gather(trn_s, [tcv, ev, lnv])
            return 0

        lax.fori_loop(0, NB, act_body, 0)

        otc = pl.ds(OTC * env, OTC)
        pltpu.sync_copy(obs_tb, obs_o.at[:, otc])
        pltpu.sync_copy(nxt_tb, nxt_o.at[:, otc])
        pltpu.sync_copy(act_tb, act_o.at[:, otc])
        rowb = pl.ds(env * BATCH, BATCH)
        pltpu.sync_copy(rew_tb, rew_o.at[rowb])
        pltpu.sync_copy(don_tb, don_o.at[rowb])
        pltpu.sync_copy(trn_tb, trn_o.at[rowb])
        return 0

    lax.fori_loop(0, E_PER, env_body, 0)


def kernel(observations, actions, rewards, dones, truncations,
           next_observations, indices):
    n_env, buf, n_obs = observations.shape
    n_act = actions.shape[-1]
    batch = indices.shape[1]
    n = n_env * batch
    idt = dones.dtype

    mesh = plsc.VectorSubcoreMesh(
        core_axis_name="c", subcore_axis_name="s",
        num_cores=NC, num_subcores=NS)
    f = pl.kernel(
        _body,
        out_type=(
            jax.ShapeDtypeStruct((OBS_BANDS, n // LANES, SUB, LANES),
                                 jnp.float32),
            jax.ShapeDtypeStruct((OBS_BANDS, n // LANES, SUB, LANES),
                                 jnp.float32),
            jax.ShapeDtypeStruct((ACT_BANDS, n // LANES, SUB, LANES),
                                 jnp.float32),
            jax.ShapeDtypeStruct((n,), jnp.float32),
            jax.ShapeDtypeStruct((n,), idt),
            jax.ShapeDtypeStruct((n,), idt),
        ),
        mesh=mesh,
        compiler_params=pltpu.CompilerParams(
            use_tc_tiling_on_sc=False, needs_layout_passes=False),
        scratch_types=[
            pltpu.VMEM((batch // LANES, SUB, LANES), jnp.int32),
            pltpu.VMEM((TCOLS, SUB, LANES), jnp.float32),
            pltpu.VMEM((TCOLS, SUB, LANES), idt),
            pltpu.VMEM((TCOLS, SUB, LANES), idt),
            pltpu.VMEM((HALF, TCOLS, SUB, LANES), jnp.float32),
            pltpu.VMEM((OBS_BANDS, OTC, SUB, LANES), jnp.float32),
            pltpu.VMEM((OBS_BANDS, OTC, SUB, LANES), jnp.float32),
            pltpu.VMEM((ACT_BANDS, OTC, SUB, LANES), jnp.float32),
            pltpu.VMEM((BATCH,), jnp.float32),
            pltpu.VMEM((BATCH,), idt),
            pltpu.VMEM((BATCH,), idt),
        ],
    )
    obs_t, nxt_t, act_t, rew, dns, trn = f(
        _tiled3(observations), _tiled3(actions),
        _tiled2(rewards), _tiled2(dones), _tiled2(truncations),
        _tiled3(next_observations), _tiled2(indices.astype(jnp.int32)))
    return (_untile_out(obs_t, n, n_obs), _untile_out(nxt_t, n, n_obs),
            _untile_out(act_t, n, n_act), rew, dns, trn)


# speedup vs baseline: 4.6257x; 1.3837x over previous
import jax
import jax.numpy as jnp
from jax import lax
from jax.experimental import pallas as pl
from jax.experimental.pallas import tpu as pltpu
from jax.experimental.pallas import tpu_sc as plsc

N_ENV = 256
BUF = 2048
N_OBS = 64
N_ACT = 16
BATCH = 256

NC = 2
NS = 16
L = 16
NW = NC * NS
E_PER = N_ENV // NW

SUB = 8
LANES = 128
TCOLS = BUF // LANES
OBS_BANDS = N_OBS // SUB
ACT_BANDS = N_ACT // SUB
HALF = 2
OTC = BATCH // LANES
NB = BATCH // L


def _tiled3(x):
    n_env, t, f = x.shape
    return (x.transpose(0, 2, 1)
             .reshape(n_env, f // SUB, SUB, t // LANES, LANES)
             .transpose(0, 1, 3, 2, 4))


def _tiled2(x):
    a, b = x.shape
    return (x.reshape(a // SUB, SUB, b // LANES, LANES)
             .transpose(0, 2, 1, 3))


def _untile_out(y, n, f):
    return y.transpose(1, 3, 0, 2).reshape(n, f)


SLABW = HALF * TCOLS * SUB * LANES
SCALW = TCOLS * SUB * LANES


def _body(obs_hbm, act_hbm, rew_hbm, don_hbm, trn_hbm, nxt_hbm, idx_hbm,
          obs_o, nxt_o, act_o, rew_o, don_o, trn_o,
          idx_s, sidx_v, slab0, slab1,
          obs_tb, nxt_tb, act_tb, rew_tb, don_tb, trn_tb, sem, sem2):
    wid = lax.axis_index("c") * NS + lax.axis_index("s")
    pltpu.sync_copy(idx_hbm.at[wid], idx_s)
    slabs = (slab0, slab1)

    def blk_idx(e, j):
        tv = idx_s[j >> 3, e, pl.ds((j & 7) * L, L)]
        return ((tv >> 7) << 10) + (tv & 127)

    def env_body(e, _):
        env = wid * E_PER + e
        srcs = [obs_hbm.at[env].at[pl.ds(h * SLABW, SLABW)]
                for h in range(OBS_BANDS // HALF)]
        srcs += [nxt_hbm.at[env].at[pl.ds(h * SLABW, SLABW)]
                 for h in range(OBS_BANDS // HALF)]
        srcs.append(act_hbm.at[env])
        nt = len(srcs)
        pltpu.async_copy(srcs[0], slabs[0], sem).wait()
        eoff = e << 7

        def sidx_body(j, _):
            sidx_v[pl.ds(j * L, L)] = blk_idx(e, j) + eoff
            return 0

        lax.fori_loop(0, NB, sidx_body, 0)
        scal_cps = []
        for c in range(BATCH // 128):
            s = pl.ds(c * 128, 128)
            ig = sidx_v.at[s]
            scal_cps.append(pltpu.async_copy(
                rew_hbm.at[wid].at[ig], rew_tb.at[s], sem2))
            scal_cps.append(pltpu.async_copy(
                don_hbm.at[wid].at[ig], don_tb.at[s], sem2))
            scal_cps.append(pltpu.async_copy(
                trn_hbm.at[wid].at[ig], trn_tb.at[s], sem2))

        for t in range(nt):
            slab = slabs[t % 2]
            cp = (pltpu.async_copy(srcs[t + 1], slabs[(t + 1) % 2], sem)
                  if t + 1 < nt else None)
            if t < 2 * (OBS_BANDS // HALF):
                dst = obs_tb if t < OBS_BANDS // HALF else nxt_tb
                h = t % (OBS_BANDS // HALF)

                def half_body(j, _, slab=slab, dst=dst, h=h):
                    base = blk_idx(e, j)
                    for fl in range(HALF * SUB):
                        fidx = base + ((fl >> 3) * 16384 + (fl & 7) * 128)
                        g = plsc.load_gather(slab, [fidx])
                        dst[h * HALF + (fl >> 3), j >> 3, fl & 7,
                            pl.ds((j & 7) * L, L)] = g
                    return 0

                lax.fori_loop(0, NB, half_body, 0)
            else:
                def act_body(j, _, slab=slab):
                    base = blk_idx(e, j)
                    for fl in range(N_ACT):
                        fidx = base + ((fl >> 3) * 16384 + (fl & 7) * 128)
                        g = plsc.load_gather(slab, [fidx])
                        act_tb[fl >> 3, j >> 3, fl & 7,
                               pl.ds((j & 7) * L, L)] = g
                    return 0

                lax.fori_loop(0, NB, act_body, 0)
            if cp is not None:
                cp.wait()
        for scp in scal_cps:
            scp.wait()

        otc = pl.ds(OTC * env, OTC)
        pltpu.sync_copy(obs_tb, obs_o.at[:, otc])
        pltpu.sync_copy(nxt_tb, nxt_o.at[:, otc])
        pltpu.sync_copy(act_tb, act_o.at[:, otc])
        rowb = pl.ds(env * BATCH, BATCH)
        pltpu.sync_copy(rew_tb, rew_o.at[rowb])
        pltpu.sync_copy(don_tb, don_o.at[rowb])
        pltpu.sync_copy(trn_tb, trn_o.at[rowb])
        return 0

    lax.fori_loop(0, E_PER, env_body, 0)


def kernel(observations, actions, rewards, dones, truncations,
           next_observations, indices):
    n_env, buf, n_obs = observations.shape
    n_act = actions.shape[-1]
    batch = indices.shape[1]
    n = n_env * batch
    idt = dones.dtype

    mesh = plsc.VectorSubcoreMesh(
        core_axis_name="c", subcore_axis_name="s",
        num_cores=NC, num_subcores=NS)
    f = pl.kernel(
        _body,
        out_type=(
            jax.ShapeDtypeStruct((OBS_BANDS, n // LANES, SUB, LANES),
                                 jnp.float32),
            jax.ShapeDtypeStruct((OBS_BANDS, n // LANES, SUB, LANES),
                                 jnp.float32),
            jax.ShapeDtypeStruct((ACT_BANDS, n // LANES, SUB, LANES),
                                 jnp.float32),
            jax.ShapeDtypeStruct((n,), jnp.float32),
            jax.ShapeDtypeStruct((n,), idt),
            jax.ShapeDtypeStruct((n,), idt),
        ),
        mesh=mesh,
        compiler_params=pltpu.CompilerParams(
            use_tc_tiling_on_sc=False, needs_layout_passes=False),
        scratch_types=[
            pltpu.VMEM((batch // LANES, SUB, LANES), jnp.int32),
            pltpu.VMEM((BATCH,), jnp.int32),
            pltpu.VMEM((SLABW,), jnp.float32),
            pltpu.VMEM((SLABW,), jnp.float32),
            pltpu.VMEM((OBS_BANDS, OTC, SUB, LANES), jnp.float32),
            pltpu.VMEM((OBS_BANDS, OTC, SUB, LANES), jnp.float32),
            pltpu.VMEM((ACT_BANDS, OTC, SUB, LANES), jnp.float32),
            pltpu.VMEM((BATCH,), jnp.float32),
            pltpu.VMEM((BATCH,), idt),
            pltpu.VMEM((BATCH,), idt),
            pltpu.SemaphoreType.DMA,
            pltpu.SemaphoreType.DMA,
        ],
    )
    obs_t, nxt_t, act_t, rew, dns, trn = f(
        _tiled3(observations).reshape(n_env, -1),
        _tiled3(actions).reshape(n_env, -1),
        _tiled2(rewards).reshape(n_env // SUB, -1),
        _tiled2(dones).reshape(n_env // SUB, -1),
        _tiled2(truncations).reshape(n_env // SUB, -1),
        _tiled3(next_observations).reshape(n_env, -1),
        _tiled2(indices.astype(jnp.int32)))
    return (_untile_out(obs_t, n, n_obs), _untile_out(nxt_t, n, n_obs),
            _untile_out(act_t, n, n_act), rew, dns, trn)


# speedup vs baseline: 4.7719x; 1.0316x over previous
import jax
import jax.numpy as jnp
from jax import lax
from jax.experimental import pallas as pl
from jax.experimental.pallas import tpu as pltpu
from jax.experimental.pallas import tpu_sc as plsc

N_ENV = 256
BUF = 2048
N_OBS = 64
N_ACT = 16
BATCH = 256

NC = 2
NS = 16
L = 16
NW = NC * NS
E_PER = N_ENV // NW

SUB = 8
LANES = 128
TCOLS = BUF // LANES
OBS_BANDS = N_OBS // SUB
ACT_BANDS = N_ACT // SUB
HALF = 2
OTC = BATCH // LANES
NB = BATCH // L
UNR = 2

SLABW = HALF * TCOLS * SUB * LANES
BANDW = TCOLS * SUB * LANES
OROW = OTC * SUB * LANES
NDENSE = 2 * (OBS_BANDS // HALF)
NT = NDENSE + ACT_BANDS


def _tiled3(x):
    n_env, t, f = x.shape
    return (x.transpose(0, 2, 1)
             .reshape(n_env, f // SUB, SUB, t // LANES, LANES)
             .transpose(0, 1, 3, 2, 4)
             .reshape(n_env, (f // SUB) * (t // LANES) * SUB * LANES))


def _tiled2(x):
    a, b = x.shape
    return (x.reshape(a // SUB, SUB, b // LANES, LANES)
             .transpose(0, 2, 1, 3)
             .reshape(a // SUB, (b // LANES) * SUB * LANES))


def _untile_out(y, n, f):
    return y.transpose(1, 3, 0, 2).reshape(n, f)


def _body(obs_hbm, act_hbm, rew_hbm, don_hbm, trn_hbm, nxt_hbm, idx_hbm,
          obs_o, nxt_o, act_o, rew_o, don_o, trn_o,
          idx_s, sidx_v, slab0, slab1,
          obs_tb, nxt_tb, act_tb, rew_tb, don_tb, trn_tb,
          sem, sem2, sem3):
    wid = lax.axis_index("c") * NS + lax.axis_index("s")
    pltpu.sync_copy(idx_hbm.at[wid], idx_s)
    slabs = (slab0, slab1)

    def blk_idx(e, j):
        tv = idx_s[pl.ds(((j >> 3) << 10) + (e << 7) + (j & 7) * L, L)]
        return ((tv >> 7) << 10) + (tv & 127)

    def srcs_for(env):
        s = [obs_hbm.at[env].at[pl.ds(h * SLABW, SLABW)]
             for h in range(OBS_BANDS // HALF)]
        s += [nxt_hbm.at[env].at[pl.ds(h * SLABW, SLABW)]
              for h in range(OBS_BANDS // HALF)]
        s += [act_hbm.at[env].at[pl.ds(b * BANDW, BANDW)]
              for b in range(ACT_BANDS)]
        return s

    def slab_dst(t):
        d = slabs[t % 2]
        return d if t < NDENSE else d.at[pl.ds(0, BANDW)]

    def drain_outs(env):
        otc = pl.ds(OTC * env, OTC)
        rowb = pl.ds(env * BATCH, BATCH)
        pltpu.make_async_copy(obs_tb, obs_o.at[:, otc], sem3).wait()
        pltpu.make_async_copy(nxt_tb, nxt_o.at[:, otc], sem3).wait()
        pltpu.make_async_copy(act_tb, act_o.at[:, otc], sem3).wait()
        pltpu.make_async_copy(rew_tb, rew_o.at[rowb], sem3).wait()
        pltpu.make_async_copy(don_tb, don_o.at[rowb], sem3).wait()
        pltpu.make_async_copy(trn_tb, trn_o.at[rowb], sem3).wait()

    pltpu.async_copy(srcs_for(wid * E_PER)[0], slabs[0], sem)

    def env_body(e, _):
        env = wid * E_PER + e
        srcs = srcs_for(env)
        eoff = e << 7

        def sidx_body(j, _):
            sidx_v[pl.ds(j * L, L)] = blk_idx(e, j) + eoff
            return 0

        lax.fori_loop(0, NB, sidx_body, 0)

        @pl.when(e != 0)
        def _():
            drain_outs(env - 1)

        scal_cps = []
        for c in range(BATCH // 128):
            s = pl.ds(c * 128, 128)
            ig = sidx_v.at[s]
            scal_cps.append(pltpu.async_copy(
                rew_hbm.at[wid].at[ig], rew_tb.at[s], sem2))
            scal_cps.append(pltpu.async_copy(
                don_hbm.at[wid].at[ig], don_tb.at[s], sem2))
            scal_cps.append(pltpu.async_copy(
                trn_hbm.at[wid].at[ig], trn_tb.at[s], sem2))

        for t in range(NT):
            pltpu.make_async_copy(srcs[t], slab_dst(t), sem).wait()
            if t + 1 < NT:
                pltpu.async_copy(srcs[t + 1], slab_dst(t + 1), sem)
            else:
                @pl.when(e != E_PER - 1)
                def _():
                    pltpu.async_copy(
                        obs_hbm.at[env + 1].at[pl.ds(0, SLABW)],
                        slabs[0], sem)
            slab = slabs[t % 2]
            if t < NDENSE:
                dst = obs_tb if t < OBS_BANDS // HALF else nxt_tb
                h = t % (OBS_BANDS // HALF)

                def dense_body(k, _, slab=slab, dst=dst, h=h):
                    for u in range(UNR):
                        j = k * UNR + u
                        base = blk_idx(e, j)
                        for fl in range(HALF * SUB):
                            fidx = base + ((fl >> 3) * BANDW
                                           + (fl & 7) * LANES)
                            g = plsc.load_gather(slab, [fidx])
                            dst[h * HALF + (fl >> 3), j >> 3, fl & 7,
                                pl.ds((j & 7) * L, L)] = g
                    return 0

                lax.fori_loop(0, NB // UNR, dense_body, 0)
            else:
                b = t - NDENSE

                def act_body(k, _, slab=slab, b=b):
                    for u in range(UNR):
                        j = k * UNR + u
                        base = blk_idx(e, j)
                        for fl in range(SUB):
                            g = plsc.load_gather(
                                slab, [base + fl * LANES])
                            act_tb[b, j >> 3, fl,
                                   pl.ds((j & 7) * L, L)] = g
                    return 0

                lax.fori_loop(0, NB // UNR, act_body, 0)

        for scp in scal_cps:
            scp.wait()
        otc = pl.ds(OTC * env, OTC)
        rowb = pl.ds(env * BATCH, BATCH)
        pltpu.async_copy(obs_tb, obs_o.at[:, otc], sem3)
        pltpu.async_copy(nxt_tb, nxt_o.at[:, otc], sem3)
        pltpu.async_copy(act_tb, act_o.at[:, otc], sem3)
        pltpu.async_copy(rew_tb, rew_o.at[rowb], sem3)
        pltpu.async_copy(don_tb, don_o.at[rowb], sem3)
        pltpu.async_copy(trn_tb, trn_o.at[rowb], sem3)
        return 0

    lax.fori_loop(0, E_PER, env_body, 0)
    drain_outs(wid * E_PER + E_PER - 1)


def kernel(observations, actions, rewards, dones, truncations,
           next_observations, indices):
    n_env, buf, n_obs = observations.shape
    n_act = actions.shape[-1]
    batch = indices.shape[1]
    n = n_env * batch
    idt = dones.dtype

    mesh = plsc.VectorSubcoreMesh(
        core_axis_name="c", subcore_axis_name="s",
        num_cores=NC, num_subcores=NS)
    f = pl.kernel(
        _body,
        out_type=(
            jax.ShapeDtypeStruct((OBS_BANDS, n // LANES, SUB, LANES),
                                 jnp.float32),
            jax.ShapeDtypeStruct((OBS_BANDS, n // LANES, SUB, LANES),
                                 jnp.float32),
            jax.ShapeDtypeStruct((ACT_BANDS, n // LANES, SUB, LANES),
                                 jnp.float32),
            jax.ShapeDtypeStruct((n,), jnp.float32),
            jax.ShapeDtypeStruct((n,), idt),
            jax.ShapeDtypeStruct((n,), idt),
        ),
        mesh=mesh,
        compiler_params=pltpu.CompilerParams(
            use_tc_tiling_on_sc=False, needs_layout_passes=False),
        scratch_types=[
            pltpu.VMEM(((batch // LANES) * SUB * LANES,), jnp.int32),
            pltpu.VMEM((BATCH,), jnp.int32),
            pltpu.VMEM((SLABW,), jnp.float32),
            pltpu.VMEM((SLABW,), jnp.float32),
            pltpu.VMEM((OBS_BANDS, OTC, SUB, LANES), jnp.float32),
            pltpu.VMEM((OBS_BANDS, OTC, SUB, LANES), jnp.float32),
            pltpu.VMEM((ACT_BANDS, OTC, SUB, LANES), jnp.float32),
            pltpu.VMEM((BATCH,), jnp.float32),
            pltpu.VMEM((BATCH,), idt),
            pltpu.VMEM((BATCH,), idt),
            pltpu.SemaphoreType.DMA,
            pltpu.SemaphoreType.DMA,
            pltpu.SemaphoreType.DMA,
        ],
    )
    obs_t, nxt_t, act_t, rew, dns, trn = f(
        _tiled3(observations), _tiled3(actions),
        _tiled2(rewards), _tiled2(dones), _tiled2(truncations),
        _tiled3(next_observations), _tiled2(indices.astype(jnp.int32)))
    return (_untile_out(obs_t, n, n_obs), _untile_out(nxt_t, n, n_obs),
            _untile_out(act_t, n, n_act), rew, dns, trn)


# speedup vs baseline: 5.7542x; 1.2059x over previous
import jax
import jax.numpy as jnp
from jax import lax
from jax.experimental import pallas as pl
from jax.experimental.pallas import tpu as pltpu
from jax.experimental.pallas import tpu_sc as plsc

N_ENV = 256
BUF = 2048
N_OBS = 64
N_ACT = 16
BATCH = 256

NC = 2
NS = 16
L = 16
NW = NC * NS
E_PER = N_ENV // NW

SUB = 8
LANES = 128
TCOLS = BUF // LANES
OBS_BANDS = N_OBS // SUB
ACT_BANDS = N_ACT // SUB
HALF = 2
OTC = BATCH // LANES
NB = BATCH // L
UNR = 2

BANDW = TCOLS * SUB * LANES
NT = 2 * OBS_BANDS + ACT_BANDS
RING = 3
DEPTH = RING - 1


def _tiled3(x):
    n_env, t, f = x.shape
    return (x.transpose(0, 2, 1)
             .reshape(n_env, f // SUB, SUB, t // LANES, LANES)
             .transpose(0, 1, 3, 2, 4)
             .reshape(n_env, (f // SUB) * (t // LANES) * SUB * LANES))


def _tiled2(x):
    a, b = x.shape
    return (x.reshape(a // SUB, SUB, b // LANES, LANES)
             .transpose(0, 2, 1, 3)
             .reshape(a // SUB, (b // LANES) * SUB * LANES))


def _untile_out(y, n, f):
    return y.transpose(1, 3, 0, 2).reshape(n, f)


def _body(obs_hbm, act_hbm, rew_hbm, don_hbm, trn_hbm, nxt_hbm, idx_hbm,
          obs_o, nxt_o, act_o, rew_o, don_o, trn_o,
          idx_s, sidx_v, slab0, slab1, slab2,
          obs_tb, nxt_tb, act_tb, rew_tb, don_tb, trn_tb,
          sem, sem2, sem3):
    wid = lax.axis_index("c") * NS + lax.axis_index("s")
    pltpu.sync_copy(idx_hbm.at[wid], idx_s)
    slabs = (slab0, slab1, slab2)

    def blk_idx(e, j):
        tv = idx_s[pl.ds(((j >> 3) << 10) + (e << 7) + (j & 7) * L, L)]
        return ((tv >> 7) << 10) + (tv & 127)

    def srcs_for(env):
        s = [obs_hbm.at[env].at[pl.ds(b * BANDW, BANDW)]
             for b in range(OBS_BANDS)]
        s += [nxt_hbm.at[env].at[pl.ds(b * BANDW, BANDW)]
              for b in range(OBS_BANDS)]
        s += [act_hbm.at[env].at[pl.ds(b * BANDW, BANDW)]
              for b in range(ACT_BANDS)]
        return s

    def drain_outs(env):
        otc = pl.ds(OTC * env, OTC)
        rowb = pl.ds(env * BATCH, BATCH)
        pltpu.make_async_copy(obs_tb, obs_o.at[:, otc], sem3).wait()
        pltpu.make_async_copy(nxt_tb, nxt_o.at[:, otc], sem3).wait()
        pltpu.make_async_copy(act_tb, act_o.at[:, otc], sem3).wait()
        pltpu.make_async_copy(rew_tb, rew_o.at[rowb], sem3).wait()
        pltpu.make_async_copy(don_tb, don_o.at[rowb], sem3).wait()
        pltpu.make_async_copy(trn_tb, trn_o.at[rowb], sem3).wait()

    for t in range(DEPTH):
        pltpu.async_copy(srcs_for(wid * E_PER)[t], slabs[t % RING], sem)

    def env_body(e, _):
        env = wid * E_PER + e
        srcs = srcs_for(env)
        eoff = e << 7

        def sidx_body(j, _):
            sidx_v[pl.ds(j * L, L)] = blk_idx(e, j) + eoff
            return 0

        lax.fori_loop(0, NB, sidx_body, 0)

        @pl.when(e != 0)
        def _():
            drain_outs(env - 1)

        scal_cps = []
        for c in range(BATCH // 128):
            s = pl.ds(c * 128, 128)
            ig = sidx_v.at[s]
            scal_cps.append(pltpu.async_copy(
                rew_hbm.at[wid].at[ig], rew_tb.at[s], sem2))
            scal_cps.append(pltpu.async_copy(
                don_hbm.at[wid].at[ig], don_tb.at[s], sem2))
            scal_cps.append(pltpu.async_copy(
                trn_hbm.at[wid].at[ig], trn_tb.at[s], sem2))

        for t in range(NT):
            slab = slabs[t % RING]
            pltpu.make_async_copy(srcs[t], slab, sem).wait()
            if t + DEPTH < NT:
                pltpu.async_copy(srcs[t + DEPTH], slabs[(t + DEPTH) % RING],
                                 sem)
            else:
                @pl.when(e != E_PER - 1)
                def _(t=t):
                    pltpu.async_copy(
                        obs_hbm.at[env + 1]
                               .at[pl.ds((t + DEPTH - NT) * BANDW, BANDW)],
                        slabs[(t + DEPTH) % RING], sem)
            if t < OBS_BANDS:
                dst, row = obs_tb, t
            elif t < 2 * OBS_BANDS:
                dst, row = nxt_tb, t - OBS_BANDS
            else:
                dst, row = act_tb, t - 2 * OBS_BANDS

            def band_body(k, _, slab=slab, dst=dst, row=row):
                for u in range(UNR):
                    j = k * UNR + u
                    base = blk_idx(e, j)
                    for fl in range(SUB):
                        g = plsc.load_gather(slab, [base + fl * LANES])
                        dst[row, j >> 3, fl, pl.ds((j & 7) * L, L)] = g
                return 0

            lax.fori_loop(0, NB // UNR, band_body, 0)

        for scp in scal_cps:
            scp.wait()
        otc = pl.ds(OTC * env, OTC)
        rowb = pl.ds(env * BATCH, BATCH)
        pltpu.async_copy(obs_tb, obs_o.at[:, otc], sem3)
        pltpu.async_copy(nxt_tb, nxt_o.at[:, otc], sem3)
        pltpu.async_copy(act_tb, act_o.at[:, otc], sem3)
        pltpu.async_copy(rew_tb, rew_o.at[rowb], sem3)
        pltpu.async_copy(don_tb, don_o.at[rowb], sem3)
        pltpu.async_copy(trn_tb, trn_o.at[rowb], sem3)
        return 0

    lax.fori_loop(0, E_PER, env_body, 0)
    drain_outs(wid * E_PER + E_PER - 1)


def kernel(observations, actions, rewards, dones, truncations,
           next_observations, indices):
    n_env, buf, n_obs = observations.shape
    n_act = actions.shape[-1]
    batch = indices.shape[1]
    n = n_env * batch
    idt = dones.dtype

    mesh = plsc.VectorSubcoreMesh(
        core_axis_name="c", subcore_axis_name="s",
        num_cores=NC, num_subcores=NS)
    f = pl.kernel(
        _body,
        out_type=(
            jax.ShapeDtypeStruct((OBS_BANDS, n // LANES, SUB, LANES),
                                 jnp.float32),
            jax.ShapeDtypeStruct((OBS_BANDS, n // LANES, SUB, LANES),
                                 jnp.float32),
            jax.ShapeDtypeStruct((ACT_BANDS, n // LANES, SUB, LANES),
                                 jnp.float32),
            jax.ShapeDtypeStruct((n,), jnp.float32),
            jax.ShapeDtypeStruct((n,), idt),
            jax.ShapeDtypeStruct((n,), idt),
        ),
        mesh=mesh,
        compiler_params=pltpu.CompilerParams(
            use_tc_tiling_on_sc=False, needs_layout_passes=False),
        scratch_types=[
            pltpu.VMEM(((batch // LANES) * SUB * LANES,), jnp.int32),
            pltpu.VMEM((BATCH,), jnp.int32),
            pltpu.VMEM((BANDW,), jnp.float32),
            pltpu.VMEM((BANDW,), jnp.float32),
            pltpu.VMEM((BANDW,), jnp.float32),
            pltpu.VMEM((OBS_BANDS, OTC, SUB, LANES), jnp.float32),
            pltpu.VMEM((OBS_BANDS, OTC, SUB, LANES), jnp.float32),
            pltpu.VMEM((ACT_BANDS, OTC, SUB, LANES), jnp.float32),
            pltpu.VMEM((BATCH,), jnp.float32),
            pltpu.VMEM((BATCH,), idt),
            pltpu.VMEM((BATCH,), idt),
            pltpu.SemaphoreType.DMA,
            pltpu.SemaphoreType.DMA,
            pltpu.SemaphoreType.DMA,
        ],
    )
    obs_t, nxt_t, act_t, rew, dns, trn = f(
        _tiled3(observations), _tiled3(actions),
        _tiled2(rewards), _tiled2(dones), _tiled2(truncations),
        _tiled3(next_observations), _tiled2(indices.astype(jnp.int32)))
    return (_untile_out(obs_t, n, n_obs), _untile_out(nxt_t, n, n_obs),
            _untile_out(act_t, n, n_act), rew, dns, trn)
